# baseline (speedup 1.0000x reference)
import functools

import jax
import jax.numpy as jnp
from jax import lax
from jax.experimental import pallas as pl
from jax.experimental.pallas import tpu as pltpu
from jax.experimental.pallas import tpu_sc as plsc

NR = 10000
NO = 50000
DIN = 128
DOUT = 16
NT = 16
K = 128
CW = 32
NCH = 4

NB = 5
_EM = NT * K * NB


def _pad_e(e):
    return ((e + _EM - 1) // _EM) * _EM


ERR, ERO, EOR, EOO = 100000, 200000, 200000, 100000
ERRP, EROP, EORP, EOOP = _pad_e(ERR), _pad_e(ERO), _pad_e(EOR), _pad_e(EOO)

R_ROOM = 10240
R_OBJ = 50176
HR_ROOM = 384
HR_OBJ = 1664


def _sc_l1_body(xr4, xo4,
                srr, drr, sro, dro, sor, dor, soo, doo,
                arr, aro, aor, aoo, crr, cro, cor, coo,
                acc, acc1, idx_s, idx_d, rows, zbuf, zb1, ones1,
                *sems):
    cid = lax.axis_index("c")
    sid = lax.axis_index("s")
    sg = sems[:NB]
    ss = sems[NB:2 * NB]
    si0, si1 = sems[2 * NB], sems[2 * NB + 1]

    @pl.loop(0, 64)
    def _(r):
        zbuf[r, pl.ds(0, 16)] = jnp.zeros((16,), jnp.float32)
        zbuf[r, pl.ds(16, 16)] = jnp.zeros((16,), jnp.float32)

    @pl.loop(0, 64, step=16)
    def _(r):
        zb1[pl.ds(r, 16)] = jnp.zeros((16,), jnp.float32)

    @pl.loop(0, K, step=16)
    def _(r):
        ones1[pl.ds(r, 16)] = jnp.ones((16,), jnp.float32)

    def zero_acc(nrows):
        rpt = nrows // NT

        @pl.loop(0, rpt, step=64)
        def _(j):
            pltpu.sync_copy(zbuf.at[pl.ds(0, 64)],
                            acc.at[pl.ds(sid * rpt + j, 64)])

        plsc.subcore_barrier()

    def edge_pass(table, cc, s2, d2, ep):
        rpt = ep // (NT * K)
        ngr = rpt // NB
        row0 = sid * rpt

        def idx_xform(base):
            for b in range(NB):
                for w in range(K // 16):
                    ds_w = pl.ds(w * 16, 16)
                    v = idx_s[base + b, ds_w]
                    idx_s[base + b, ds_w] = v * 4 + cc

        def idx_start(g, p):
            r = row0 + g * NB
            q = p * NB
            pltpu.async_copy(s2.at[pl.ds(r, NB)],
                             idx_s.at[pl.ds(q, NB)], si0)
            pltpu.async_copy(d2.at[pl.ds(r, NB)],
                             idx_d.at[pl.ds(q, NB)], si1)

        def idx_wait(g, p):
            r = row0 + g * NB
            q = p * NB
            pltpu.make_async_copy(s2.at[pl.ds(r, NB)],
                                  idx_s.at[pl.ds(q, NB)], si0).wait()
            pltpu.make_async_copy(d2.at[pl.ds(r, NB)],
                                  idx_d.at[pl.ds(q, NB)], si1).wait()

        idx_start(0, 0)
        idx_wait(0, 0)
        idx_start(1, 1)
        idx_xform(0)
        gd = [pltpu.async_copy(table.at[idx_s.at[b]], rows.at[b], sg[b])
              for b in range(NB)]
        for b in range(NB):
            gd[b].wait()
            pltpu.async_copy(rows.at[b], acc.at[idx_d.at[b]], ss[b],
                             add=True)

        @pl.loop(1, ngr)
        def _(g):
            p = lax.rem(g, 2)
            base = p * NB
            idx_wait(g, p)
            idx_xform(base)
            for b in range(NB):
                pltpu.make_async_copy(rows.at[b], acc.at[idx_d.at[base + b]],
                                      ss[b]).wait()

            @pl.when(g + 1 < ngr)
            def _():
                idx_start(g + 1, 1 - p)

            gd = [pltpu.async_copy(table.at[idx_s.at[base + b]], rows.at[b],
                                   sg[b])
                  for b in range(NB)]
            for b in range(NB):
                gd[b].wait()
                pltpu.async_copy(rows.at[b], acc.at[idx_d.at[base + b]],
                                 ss[b], add=True)

        for b in range(NB):
            pltpu.make_async_copy(rows.at[b], acc.at[idx_d.at[b]],
                                  ss[b]).wait()
        plsc.subcore_barrier()

    def count_pass(d2, ep, nwords, out_ref):
        zpt = nwords // NT

        @pl.loop(0, zpt, step=64)
        def _(j):
            pltpu.sync_copy(zb1, acc1.at[pl.ds(sid * zpt + j, 64)])

        plsc.subcore_barrier()
        rpt = ep // (NT * K)
        row0 = sid * rpt

        @pl.loop(0, rpt // NB)
        def _(g):
            pltpu.sync_copy(d2.at[pl.ds(row0 + g * NB, NB)],
                            idx_d.at[pl.ds(0, NB)])
            sd = [pltpu.async_copy(ones1, acc1.at[idx_d.at[b]], ss[b],
                                   add=True)
                  for b in range(NB)]
            for d in sd:
                d.wait()

        plsc.subcore_barrier()
        ds_ = pl.ds(sid * zpt, zpt)
        pltpu.sync_copy(acc1.at[ds_], out_ref.at[ds_])
        plsc.subcore_barrier()

    def flush(out_ref, c, n):
        rpf = n // NT
        ds = pl.ds(sid * rpf, rpf)
        if c is None:
            pltpu.sync_copy(acc.at[ds], out_ref.at[ds])
        else:
            pltpu.sync_copy(acc.at[ds], out_ref.at[c, ds])
        plsc.subcore_barrier()

    @pl.when(cid == 0)
    def _():
        for c in range(NCH):
            zero_acc(R_ROOM)
            edge_pass(xr4, c, srr, drr, ERRP)
            flush(arr, c, R_ROOM)
        for c in range(NCH):
            zero_acc(R_OBJ)
            edge_pass(xr4, c, sro, dro, EROP)
            flush(aro, c, R_OBJ)
        count_pass(drr, ERRP, R_ROOM, crr)
        count_pass(dro, EROP, R_OBJ, cro)

    @pl.when(cid == 1)
    def _():
        for c in range(NCH):
            zero_acc(R_ROOM)
            edge_pass(xo4, c, sor, dor, EORP)
            flush(aor, c, R_ROOM)
        for c in range(NCH):
            zero_acc(R_OBJ)
            edge_pass(xo4, c, soo, doo, EOOP)
            flush(aoo, c, R_OBJ)
        count_pass(dor, EORP, R_ROOM, cor)
        count_pass(doo, EOOP, R_OBJ, coo)


@functools.lru_cache(maxsize=None)
def _sc_l1(interpret=False):
    f32 = jnp.float32
    return pl.kernel(
        _sc_l1_body,
        out_type=(
            jax.ShapeDtypeStruct((NCH, R_ROOM, CW), f32),
            jax.ShapeDtypeStruct((NCH, R_OBJ, CW), f32),
            jax.ShapeDtypeStruct((NCH, R_ROOM, CW), f32),
            jax.ShapeDtypeStruct((NCH, R_OBJ, CW), f32),
            jax.ShapeDtypeStruct((R_ROOM,), f32),
            jax.ShapeDtypeStruct((R_OBJ,), f32),
            jax.ShapeDtypeStruct((R_ROOM,), f32),
            jax.ShapeDtypeStruct((R_OBJ,), f32),
        ),
        mesh=plsc.VectorSubcoreMesh(core_axis_name="c", subcore_axis_name="s",
                                    num_cores=2, num_subcores=NT),
        compiler_params=pltpu.CompilerParams(use_tc_tiling_on_sc=False,
                                             needs_layout_passes=False),
        scratch_types=[
            pltpu.VMEM_SHARED((R_OBJ, CW), f32),
            pltpu.VMEM_SHARED((R_OBJ,), f32),
            pltpu.VMEM((2 * NB, K), jnp.int32),
            pltpu.VMEM((2 * NB, K), jnp.int32),
            pltpu.VMEM((NB, K, CW), f32),
            pltpu.VMEM((64, CW), f32),
            pltpu.VMEM((64,), f32),
            pltpu.VMEM((K,), f32),
        ] + [pltpu.SemaphoreType.DMA] * (2 * NB + 2),
        interpret=interpret,
    )


def _sc_l2_body(z2rr, z2or, srr, drr, sor, dor,
                a2rr, a2or,
                acc, idx_s, idx_d, rows, zbuf,
                *sems):
    cid = lax.axis_index("c")
    sid = lax.axis_index("s")
    sg = sems[:NB]
    ss = sems[NB:2 * NB]

    @pl.loop(0, 64)
    def _(r):
        zbuf[r, pl.ds(0, 16)] = jnp.zeros((16,), jnp.float32)

    def do_pass(table, s2, d2, ep, out_ref):
        rpt0 = R_ROOM // NT

        @pl.loop(0, rpt0, step=64)
        def _(j):
            pltpu.sync_copy(zbuf, acc.at[pl.ds(sid * rpt0 + j, 64)])

        plsc.subcore_barrier()
        rpt = ep // (NT * K)
        row0 = sid * rpt

        @pl.loop(0, rpt // NB)
        def _(g):
            r = row0 + g * NB
            pltpu.sync_copy(s2.at[pl.ds(r, NB)], idx_s)
            pltpu.sync_copy(d2.at[pl.ds(r, NB)], idx_d)
            gd = [pltpu.async_copy(table.at[idx_s.at[b]], rows.at[b], sg[b])
                  for b in range(NB)]
            sd = []
            for b in range(NB):
                gd[b].wait()
                sd.append(pltpu.async_copy(rows.at[b], acc.at[idx_d.at[b]],
                                           ss[b], add=True))
            for d in sd:
                d.wait()

        plsc.subcore_barrier()
        rpf = R_ROOM // NT
        ds = pl.ds(sid * rpf, rpf)
        pltpu.sync_copy(acc.at[ds], out_ref.at[ds])
        plsc.subcore_barrier()

    @pl.when(cid == 0)
    def _():
        do_pass(z2rr, srr, drr, ERRP, a2rr)

    @pl.when(cid == 1)
    def _():
        do_pass(z2or, sor, dor, EORP, a2or)


@functools.lru_cache(maxsize=None)
def _sc_l2(interpret=False):
    f32 = jnp.float32
    return pl.kernel(
        _sc_l2_body,
        out_type=(
            jax.ShapeDtypeStruct((R_ROOM, DOUT), f32),
            jax.ShapeDtypeStruct((R_ROOM, DOUT), f32),
        ),
        mesh=plsc.VectorSubcoreMesh(core_axis_name="c", subcore_axis_name="s",
                                    num_cores=2, num_subcores=NT),
        compiler_params=pltpu.CompilerParams(use_tc_tiling_on_sc=False,
                                             needs_layout_passes=False),
        scratch_types=[
            pltpu.VMEM_SHARED((R_ROOM, DOUT), f32),
            pltpu.VMEM((NB, K), jnp.int32),
            pltpu.VMEM((NB, K), jnp.int32),
            pltpu.VMEM((NB, K, DOUT), f32),
            pltpu.VMEM((64, DOUT), f32),
        ] + [pltpu.SemaphoreType.DMA] * (2 * NB),
        interpret=interpret,
    )


def _dot(a, b):
    return lax.dot_general(a, b, (((1,), (0,)), ((), ())),
                           precision=lax.Precision.HIGHEST,
                           preferred_element_type=jnp.float32)


def _t1r_body(arr, aor, crr, cor, xr,
              wn_rr, wn_or, wr_rr, wr_or, b_rr, b_or,
              wn2_rr, wr2_rr, wr2_or, b2_rr, b2_or,
              z2_out, root2_out):
    a_rr = arr[...]
    a_or = aor[...]
    m_rr = jnp.concatenate([a_rr[c] for c in range(NCH)], axis=1)
    m_or = jnp.concatenate([a_or[c] for c in range(NCH)], axis=1)
    s_rr = _dot(m_rr, wn_rr[...])
    s_or = _dot(m_or, wn_or[...])
    inv_rr = 1.0 / jnp.maximum(crr[...], 1.0)
    inv_or = 1.0 / jnp.maximum(cor[...], 1.0)
    r1 = s_rr * inv_rr + s_or * inv_or
    r1 = r1 + _dot(xr[...], wr_rr[...] + wr_or[...]) + b_rr[...] + b_or[...]
    r1 = jnp.maximum(r1, 0.0)
    z2_out[...] = _dot(r1, wn2_rr[...])
    root2_out[...] = _dot(r1, wr2_rr[...] + wr2_or[...]) + b2_rr[...] + b2_or[...]


def _t1o_body(aro, aoo, cro, coo, xo,
              wn_ro, wn_oo, wr_ro, wr_oo, b_ro, b_oo,
              wn2_or,
              z2_out, abuf_ro, abuf_oo, sem_a, sem_b):
    i = pl.program_id(0)
    R = abuf_ro.shape[1]
    ca = pltpu.make_async_copy(aro.at[:, pl.ds(i * R, R), :], abuf_ro, sem_a)
    cb = pltpu.make_async_copy(aoo.at[:, pl.ds(i * R, R), :], abuf_oo, sem_b)
    ca.start()
    cb.start()
    ca.wait()
    cb.wait()
    a_ro = abuf_ro[...]
    a_oo = abuf_oo[...]
    m_ro = jnp.concatenate([a_ro[c] for c in range(NCH)], axis=1)
    m_oo = jnp.concatenate([a_oo[c] for c in range(NCH)], axis=1)
    s_ro = _dot(m_ro, wn_ro[...])
    s_oo = _dot(m_oo, wn_oo[...])
    inv_ro = 1.0 / jnp.maximum(cro[...], 1.0)
    inv_oo = 1.0 / jnp.maximum(coo[...], 1.0)
    o1 = s_ro * inv_ro + s_oo * inv_oo
    o1 = o1 + _dot(xo[...], wr_ro[...] + wr_oo[...]) + b_ro[...] + b_oo[...]
    o1 = jnp.maximum(o1, 0.0)
    z2_out[...] = _dot(o1, wn2_or[...])


def _t2_body(a2rr, a2or, crr, cor, root2, out):
    inv_rr = 1.0 / jnp.maximum(crr[...], 1.0)
    inv_or = 1.0 / jnp.maximum(cor[...], 1.0)
    out[...] = a2rr[...] * inv_rr + a2or[...] * inv_or + root2[...]


def _full(shape):
    return pl.BlockSpec(shape, lambda i: tuple(0 for _ in shape))


@functools.lru_cache(maxsize=None)
def _t1r(interpret=False):
    f32 = jnp.float32
    R = 2000
    return pl.pallas_call(
        _t1r_body,
        grid=(NR // R,),
        in_specs=[
            pl.BlockSpec((NCH, R, CW), lambda i: (0, i, 0)),
            pl.BlockSpec((NCH, R, CW), lambda i: (0, i, 0)),
            pl.BlockSpec((R, 1), lambda i: (i, 0)),
            pl.BlockSpec((R, 1), lambda i: (i, 0)),
            pl.BlockSpec((R, DIN), lambda i: (i, 0)),
            _full((DIN, DIN)), _full((DIN, DIN)),
            _full((DIN, DIN)), _full((DIN, DIN)),
            _full((1, DIN)), _full((1, DIN)),
            _full((DIN, DOUT)), _full((DIN, DOUT)), _full((DIN, DOUT)),
            _full((1, DOUT)), _full((1, DOUT)),
        ],
        out_specs=[
            pl.BlockSpec((R, DOUT), lambda i: (i, 0)),
            pl.BlockSpec((R, DOUT), lambda i: (i, 0)),
        ],
        out_shape=[
            jax.ShapeDtypeStruct((NR, DOUT), f32),
            jax.ShapeDtypeStruct((NR, DOUT), f32),
        ],
        interpret=interpret,
    )


@functools.lru_cache(maxsize=None)
def _t1o(interpret=False):
    f32 = jnp.float32
    R = 2000
    return pl.pallas_call(
        _t1o_body,
        grid=(NO // R,),
        in_specs=[
            pl.BlockSpec(memory_space=pl.ANY),
            pl.BlockSpec(memory_space=pl.ANY),
            pl.BlockSpec((R, 1), lambda i: (i, 0)),
            pl.BlockSpec((R, 1), lambda i: (i, 0)),
            pl.BlockSpec((R, DIN), lambda i: (i, 0)),
            _full((DIN, DIN)), _full((DIN, DIN)),
            _full((DIN, DIN)), _full((DIN, DIN)),
            _full((1, DIN)), _full((1, DIN)),
            _full((DIN, DOUT)),
        ],
        out_specs=[pl.BlockSpec((R, DOUT), lambda i: (i, 0))],
        out_shape=[jax.ShapeDtypeStruct((NO, DOUT), f32)],
        scratch_shapes=[
            pltpu.VMEM((NCH, R, CW), f32),
            pltpu.VMEM((NCH, R, CW), f32),
            pltpu.SemaphoreType.DMA,
            pltpu.SemaphoreType.DMA,
        ],
        interpret=interpret,
    )


@functools.lru_cache(maxsize=None)
def _t2(interpret=False):
    f32 = jnp.float32
    R = 2000
    return pl.pallas_call(
        _t2_body,
        grid=(NR // R,),
        in_specs=[
            pl.BlockSpec((R, DOUT), lambda i: (i, 0)),
            pl.BlockSpec((R, DOUT), lambda i: (i, 0)),
            pl.BlockSpec((R, 1), lambda i: (i, 0)),
            pl.BlockSpec((R, 1), lambda i: (i, 0)),
            pl.BlockSpec((R, DOUT), lambda i: (i, 0)),
        ],
        out_specs=[pl.BlockSpec((R, DOUT), lambda i: (i, 0))],
        out_shape=[jax.ShapeDtypeStruct((NR, DOUT), f32)],
        interpret=interpret,
    )


_INTERPRET = False


def kernel(x_rooms, x_objects,
           src_rr, dst_rr, src_ro, dst_ro, src_or, dst_or, src_oo, dst_oo,
           Wn1_rr, Wr1_rr, b1_rr, Wn1_ro, Wr1_ro, b1_ro,
           Wn1_or, Wr1_or, b1_or, Wn1_oo, Wr1_oo, b1_oo,
           Wn2_rr, Wr2_rr, b2_rr, Wn2_ro, Wr2_ro, b2_ro,
           Wn2_or, Wr2_or, b2_or, Wn2_oo, Wr2_oo, b2_oo):
    it = _INTERPRET
    xr4 = x_rooms.reshape(NR * NCH, CW)
    xo4 = x_objects.reshape(NO * NCH, CW)

    def pad(s, d, ep, dummy):
        pe = ep - s.shape[0]
        s2 = jnp.concatenate([s, jnp.zeros((pe,), jnp.int32)])
        d2 = jnp.concatenate([d, jnp.full((pe,), dummy, jnp.int32)])
        return s2, d2

    def pad2(s_, d_, ep, dummy):
        s2, d2 = pad(s_, d_, ep, dummy)
        return s2.reshape(-1, K), d2.reshape(-1, K)

    srr, drr = pad2(src_rr, dst_rr, ERRP, NR)
    sro, dro = pad2(src_ro, dst_ro, EROP, NO)
    sor, dor = pad2(src_or, dst_or, EORP, NR)
    soo, doo = pad2(src_oo, dst_oo, EOOP, NO)

    arr, aro, aor, aoo, crr, cro, cor, coo = _sc_l1(it)(
        xr4, xo4, srr, drr, sro, dro, sor, dor, soo, doo)
    crr = crr[:NR].reshape(NR, 1)
    cor = cor[:NR].reshape(NR, 1)
    cro = cro[:NO].reshape(NO, 1)
    coo = coo[:NO].reshape(NO, 1)

    z2rr, root2 = _t1r(it)(
        arr, aor, crr, cor, x_rooms,
        Wn1_rr, Wn1_or, Wr1_rr, Wr1_or,
        b1_rr.reshape(1, -1), b1_or.reshape(1, -1),
        Wn2_rr, Wr2_rr, Wr2_or,
        b2_rr.reshape(1, -1), b2_or.reshape(1, -1))

    [z2or] = _t1o(it)(
        aro, aoo, cro, coo, x_objects,
        Wn1_ro, Wn1_oo, Wr1_ro, Wr1_oo,
        b1_ro.reshape(1, -1), b1_oo.reshape(1, -1),
        Wn2_or)

    a2rr, a2or = _sc_l2(it)(z2rr, z2or, srr, drr, sor, dor)

    [out] = _t2(it)(a2rr, a2or, crr, cor, root2)
    return out


# speedup vs baseline: 1.0943x; 1.0943x over previous
import functools

import jax
import jax.numpy as jnp
from jax import lax
from jax.experimental import pallas as pl
from jax.experimental.pallas import tpu as pltpu
from jax.experimental.pallas import tpu_sc as plsc

NR = 10000
NO = 50000
DIN = 128
DOUT = 16
NT = 16
K = 128
CW = 32
NCH = 4

NB = 5
_EM = NT * K * NB


def _pad_e(e):
    return ((e + _EM - 1) // _EM) * _EM


ERR, ERO, EOR, EOO = 100000, 200000, 200000, 100000
ERRP, EROP, EORP, EOOP = _pad_e(ERR), _pad_e(ERO), _pad_e(EOR), _pad_e(EOO)

R_ROOM = 10240
R_OBJ = 50176
HR_ROOM = 384
HR_OBJ = 1664


def _sc_l1_body(xr4, xo4,
                srr, drr, sro, dro, sor, dor, soo, doo,
                arr, aro, aor, aoo, crr, cro, cor, coo,
                acc, acc1, idx_s, idx_d, rows, zbuf, zb1, ones1,
                *sems):
    cid = lax.axis_index("c")
    sid = lax.axis_index("s")
    sg = sems[:NB]
    ss = sems[NB:2 * NB]
    si0, si1 = sems[2 * NB], sems[2 * NB + 1]

    @pl.loop(0, 64)
    def _(r):
        zbuf[r, pl.ds(0, 16)] = jnp.zeros((16,), jnp.float32)
        zbuf[r, pl.ds(16, 16)] = jnp.zeros((16,), jnp.float32)

    @pl.loop(0, 64, step=16)
    def _(r):
        zb1[pl.ds(r, 16)] = jnp.zeros((16,), jnp.float32)

    @pl.loop(0, K, step=16)
    def _(r):
        ones1[pl.ds(r, 16)] = jnp.ones((16,), jnp.float32)

    def zero_acc(nrows):
        rpt = nrows // NT

        @pl.loop(0, rpt, step=64)
        def _(j):
            pltpu.sync_copy(zbuf.at[pl.ds(0, 64)],
                            acc.at[pl.ds(sid * rpt + j, 64)])

        plsc.subcore_barrier()

    def edge_pass(table, cc, s2, d2, ep):
        rpt = ep // (NT * K)
        ngr = rpt // NB
        row0 = sid * rpt

        def idx_xform(base):
            for b in range(NB):
                for w in range(K // 16):
                    ds_w = pl.ds(w * 16, 16)
                    v = idx_s[base + b, ds_w]
                    idx_s[base + b, ds_w] = v * 4 + cc

        def idx_start(g, p):
            r = row0 + g * NB
            q = p * NB
            pltpu.async_copy(s2.at[pl.ds(r, NB)],
                             idx_s.at[pl.ds(q, NB)], si0)
            pltpu.async_copy(d2.at[pl.ds(r, NB)],
                             idx_d.at[pl.ds(q, NB)], si1)

        def idx_wait(g, p):
            r = row0 + g * NB
            q = p * NB
            pltpu.make_async_copy(s2.at[pl.ds(r, NB)],
                                  idx_s.at[pl.ds(q, NB)], si0).wait()
            pltpu.make_async_copy(d2.at[pl.ds(r, NB)],
                                  idx_d.at[pl.ds(q, NB)], si1).wait()

        idx_start(0, 0)
        idx_wait(0, 0)
        idx_start(1, 1)
        idx_xform(0)
        gd = [pltpu.async_copy(table.at[idx_s.at[b]], rows.at[b], sg[b])
              for b in range(NB)]
        for b in range(NB):
            gd[b].wait()
            pltpu.async_copy(rows.at[b], acc.at[idx_d.at[b]], ss[b],
                             add=True)

        @pl.loop(1, ngr)
        def _(g):
            p = lax.rem(g, 2)
            base = p * NB
            idx_wait(g, p)
            idx_xform(base)
            for b in range(NB):
                pltpu.make_async_copy(rows.at[b], acc.at[idx_d.at[base + b]],
                                      ss[b]).wait()

            @pl.when(g + 1 < ngr)
            def _():
                idx_start(g + 1, 1 - p)

            gd = [pltpu.async_copy(table.at[idx_s.at[base + b]], rows.at[b],
                                   sg[b])
                  for b in range(NB)]
            for b in range(NB):
                gd[b].wait()
                pltpu.async_copy(rows.at[b], acc.at[idx_d.at[base + b]],
                                 ss[b], add=True)

        for b in range(NB):
            pltpu.make_async_copy(rows.at[b], acc.at[idx_d.at[b]],
                                  ss[b]).wait()
        plsc.subcore_barrier()

    def count_pass(d2, ep, nwords, out_ref):
        zpt = nwords // NT

        @pl.loop(0, zpt, step=64)
        def _(j):
            pltpu.sync_copy(zb1, acc1.at[pl.ds(sid * zpt + j, 64)])

        plsc.subcore_barrier()
        rpt = ep // (NT * K)
        row0 = sid * rpt

        @pl.loop(0, rpt // NB)
        def _(g):
            pltpu.sync_copy(d2.at[pl.ds(row0 + g * NB, NB)],
                            idx_d.at[pl.ds(0, NB)])
            sd = [pltpu.async_copy(ones1, acc1.at[idx_d.at[b]], ss[b],
                                   add=True)
                  for b in range(NB)]
            for d in sd:
                d.wait()

        plsc.subcore_barrier()
        ds_ = pl.ds(sid * zpt, zpt)
        pltpu.sync_copy(acc1.at[ds_], out_ref.at[ds_])
        plsc.subcore_barrier()

    def flush(out_ref, c, n):
        rpf = n // NT
        ds = pl.ds(sid * rpf, rpf)
        if c is None:
            pltpu.sync_copy(acc.at[ds], out_ref.at[ds])
        else:
            pltpu.sync_copy(acc.at[ds], out_ref.at[c, ds])
        plsc.subcore_barrier()

    @pl.when(cid == 0)
    def _():
        for c in range(NCH):
            zero_acc(R_ROOM)
            edge_pass(xr4, c, srr, drr, ERRP)
            flush(arr, c, R_ROOM)
        for c in range(NCH):
            zero_acc(R_OBJ)
            edge_pass(xr4, c, sro, dro, EROP)
            flush(aro, c, R_OBJ)
        count_pass(drr, ERRP, R_ROOM, crr)
        count_pass(dro, EROP, R_OBJ, cro)

    @pl.when(cid == 1)
    def _():
        for c in range(NCH):
            zero_acc(R_ROOM)
            edge_pass(xo4, c, sor, dor, EORP)
            flush(aor, c, R_ROOM)
        for c in range(NCH):
            zero_acc(R_OBJ)
            edge_pass(xo4, c, soo, doo, EOOP)
            flush(aoo, c, R_OBJ)
        count_pass(dor, EORP, R_ROOM, cor)
        count_pass(doo, EOOP, R_OBJ, coo)


@functools.lru_cache(maxsize=None)
def _sc_l1(interpret=False):
    f32 = jnp.float32
    return pl.kernel(
        _sc_l1_body,
        out_type=(
            jax.ShapeDtypeStruct((NCH, R_ROOM, CW), f32),
            jax.ShapeDtypeStruct((NCH, R_OBJ, CW), f32),
            jax.ShapeDtypeStruct((NCH, R_ROOM, CW), f32),
            jax.ShapeDtypeStruct((NCH, R_OBJ, CW), f32),
            jax.ShapeDtypeStruct((R_ROOM,), f32),
            jax.ShapeDtypeStruct((R_OBJ,), f32),
            jax.ShapeDtypeStruct((R_ROOM,), f32),
            jax.ShapeDtypeStruct((R_OBJ,), f32),
        ),
        mesh=plsc.VectorSubcoreMesh(core_axis_name="c", subcore_axis_name="s",
                                    num_cores=2, num_subcores=NT),
        compiler_params=pltpu.CompilerParams(use_tc_tiling_on_sc=False,
                                             needs_layout_passes=False),
        scratch_types=[
            pltpu.VMEM_SHARED((R_OBJ, CW), f32),
            pltpu.VMEM_SHARED((R_OBJ,), f32),
            pltpu.VMEM((2 * NB, K), jnp.int32),
            pltpu.VMEM((2 * NB, K), jnp.int32),
            pltpu.VMEM((NB, K, CW), f32),
            pltpu.VMEM((64, CW), f32),
            pltpu.VMEM((64,), f32),
            pltpu.VMEM((K,), f32),
        ] + [pltpu.SemaphoreType.DMA] * (2 * NB + 2),
        interpret=interpret,
    )


def _sc_l2_body(z2rr, z2or, srr, drr, sor, dor,
                a2rr, a2or,
                acc, idx_s, idx_d, rows, zbuf,
                *sems):
    cid = lax.axis_index("c")
    sid = lax.axis_index("s")
    sg = sems[:NB]
    ss = sems[NB:2 * NB]

    @pl.loop(0, 64)
    def _(r):
        zbuf[r, pl.ds(0, 16)] = jnp.zeros((16,), jnp.float32)

    def do_pass(table, s2, d2, ep, out_ref):
        rpt0 = R_ROOM // NT

        @pl.loop(0, rpt0, step=64)
        def _(j):
            pltpu.sync_copy(zbuf, acc.at[pl.ds(sid * rpt0 + j, 64)])

        plsc.subcore_barrier()
        rpt = ep // (NT * K)
        row0 = sid * rpt

        @pl.loop(0, rpt // NB)
        def _(g):
            r = row0 + g * NB
            pltpu.sync_copy(s2.at[pl.ds(r, NB)], idx_s)
            pltpu.sync_copy(d2.at[pl.ds(r, NB)], idx_d)
            gd = [pltpu.async_copy(table.at[idx_s.at[b]], rows.at[b], sg[b])
                  for b in range(NB)]
            sd = []
            for b in range(NB):
                gd[b].wait()
                sd.append(pltpu.async_copy(rows.at[b], acc.at[idx_d.at[b]],
                                           ss[b], add=True))
            for d in sd:
                d.wait()

        plsc.subcore_barrier()
        rpf = R_ROOM // NT
        ds = pl.ds(sid * rpf, rpf)
        pltpu.sync_copy(acc.at[ds], out_ref.at[ds])
        plsc.subcore_barrier()

    @pl.when(cid == 0)
    def _():
        do_pass(z2rr, srr, drr, ERRP, a2rr)

    @pl.when(cid == 1)
    def _():
        do_pass(z2or, sor, dor, EORP, a2or)


@functools.lru_cache(maxsize=None)
def _sc_l2(interpret=False):
    f32 = jnp.float32
    return pl.kernel(
        _sc_l2_body,
        out_type=(
            jax.ShapeDtypeStruct((R_ROOM, DOUT), f32),
            jax.ShapeDtypeStruct((R_ROOM, DOUT), f32),
        ),
        mesh=plsc.VectorSubcoreMesh(core_axis_name="c", subcore_axis_name="s",
                                    num_cores=2, num_subcores=NT),
        compiler_params=pltpu.CompilerParams(use_tc_tiling_on_sc=False,
                                             needs_layout_passes=False),
        scratch_types=[
            pltpu.VMEM_SHARED((R_ROOM, DOUT), f32),
            pltpu.VMEM((NB, K), jnp.int32),
            pltpu.VMEM((NB, K), jnp.int32),
            pltpu.VMEM((NB, K, DOUT), f32),
            pltpu.VMEM((64, DOUT), f32),
        ] + [pltpu.SemaphoreType.DMA] * (2 * NB),
        interpret=interpret,
    )


def _dot(a, b):
    return lax.dot_general(a, b, (((1,), (0,)), ((), ())),
                           precision=lax.Precision.HIGHEST,
                           preferred_element_type=jnp.float32)


def _t1r_body(arr, aor, crr, cor, xr,
              wn_rr, wn_or, wr_rr, wr_or, b_rr, b_or,
              wn2_rr, wr2_rr, wr2_or, b2_rr, b2_or,
              z2_out, root2_out):
    a_rr = arr[...]
    a_or = aor[...]
    m_rr = jnp.concatenate([a_rr[c] for c in range(NCH)], axis=1)
    m_or = jnp.concatenate([a_or[c] for c in range(NCH)], axis=1)
    s_rr = _dot(m_rr, wn_rr[...])
    s_or = _dot(m_or, wn_or[...])
    inv_rr = 1.0 / jnp.maximum(crr[...], 1.0)
    inv_or = 1.0 / jnp.maximum(cor[...], 1.0)
    r1 = s_rr * inv_rr + s_or * inv_or
    r1 = r1 + _dot(xr[...], wr_rr[...] + wr_or[...]) + b_rr[...] + b_or[...]
    r1 = jnp.maximum(r1, 0.0)
    z2_out[...] = _dot(r1, wn2_rr[...])
    root2_out[...] = _dot(r1, wr2_rr[...] + wr2_or[...]) + b2_rr[...] + b2_or[...]


def _t1o_body(aro, aoo, cro, coo, xo,
              wn_ro, wn_oo, wr_ro, wr_oo, b_ro, b_oo,
              wn2_or,
              z2_out):
    a_ro = aro[...]
    a_oo = aoo[...]
    m_ro = jnp.concatenate([a_ro[c] for c in range(NCH)], axis=1)
    m_oo = jnp.concatenate([a_oo[c] for c in range(NCH)], axis=1)
    s_ro = _dot(m_ro, wn_ro[...])
    s_oo = _dot(m_oo, wn_oo[...])
    inv_ro = 1.0 / jnp.maximum(cro[...], 1.0)
    inv_oo = 1.0 / jnp.maximum(coo[...], 1.0)
    o1 = s_ro * inv_ro + s_oo * inv_oo
    o1 = o1 + _dot(xo[...], wr_ro[...] + wr_oo[...]) + b_ro[...] + b_oo[...]
    o1 = jnp.maximum(o1, 0.0)
    z2_out[...] = _dot(o1, wn2_or[...])


def _t2_body(a2rr, a2or, crr, cor, root2, out):
    inv_rr = 1.0 / jnp.maximum(crr[...], 1.0)
    inv_or = 1.0 / jnp.maximum(cor[...], 1.0)
    out[...] = a2rr[...] * inv_rr + a2or[...] * inv_or + root2[...]


def _full(shape):
    return pl.BlockSpec(shape, lambda i: tuple(0 for _ in shape))


@functools.lru_cache(maxsize=None)
def _t1r(interpret=False):
    f32 = jnp.float32
    R = 2000
    return pl.pallas_call(
        _t1r_body,
        grid=(NR // R,),
        in_specs=[
            pl.BlockSpec((NCH, R, CW), lambda i: (0, i, 0)),
            pl.BlockSpec((NCH, R, CW), lambda i: (0, i, 0)),
            pl.BlockSpec((R, 1), lambda i: (i, 0)),
            pl.BlockSpec((R, 1), lambda i: (i, 0)),
            pl.BlockSpec((R, DIN), lambda i: (i, 0)),
            _full((DIN, DIN)), _full((DIN, DIN)),
            _full((DIN, DIN)), _full((DIN, DIN)),
            _full((1, DIN)), _full((1, DIN)),
            _full((DIN, DOUT)), _full((DIN, DOUT)), _full((DIN, DOUT)),
            _full((1, DOUT)), _full((1, DOUT)),
        ],
        out_specs=[
            pl.BlockSpec((R, DOUT), lambda i: (i, 0)),
            pl.BlockSpec((R, DOUT), lambda i: (i, 0)),
        ],
        out_shape=[
            jax.ShapeDtypeStruct((NR, DOUT), f32),
            jax.ShapeDtypeStruct((NR, DOUT), f32),
        ],
        interpret=interpret,
    )


@functools.lru_cache(maxsize=None)
def _t1o(interpret=False):
    f32 = jnp.float32
    R = 2000
    return pl.pallas_call(
        _t1o_body,
        grid=(NO // R,),
        in_specs=[
            pl.BlockSpec((NCH, R, CW), lambda i: (0, i, 0)),
            pl.BlockSpec((NCH, R, CW), lambda i: (0, i, 0)),
            pl.BlockSpec((R, 1), lambda i: (i, 0)),
            pl.BlockSpec((R, 1), lambda i: (i, 0)),
            pl.BlockSpec((R, DIN), lambda i: (i, 0)),
            _full((DIN, DIN)), _full((DIN, DIN)),
            _full((DIN, DIN)), _full((DIN, DIN)),
            _full((1, DIN)), _full((1, DIN)),
            _full((DIN, DOUT)),
        ],
        out_specs=[pl.BlockSpec((R, DOUT), lambda i: (i, 0))],
        out_shape=[jax.ShapeDtypeStruct((NO, DOUT), f32)],
        interpret=interpret,
    )


@functools.lru_cache(maxsize=None)
def _t2(interpret=False):
    f32 = jnp.float32
    R = 2000
    return pl.pallas_call(
        _t2_body,
        grid=(NR // R,),
        in_specs=[
            pl.BlockSpec((R, DOUT), lambda i: (i, 0)),
            pl.BlockSpec((R, DOUT), lambda i: (i, 0)),
            pl.BlockSpec((R, 1), lambda i: (i, 0)),
            pl.BlockSpec((R, 1), lambda i: (i, 0)),
            pl.BlockSpec((R, DOUT), lambda i: (i, 0)),
        ],
        out_specs=[pl.BlockSpec((R, DOUT), lambda i: (i, 0))],
        out_shape=[jax.ShapeDtypeStruct((NR, DOUT), f32)],
        interpret=interpret,
    )


def kernel(x_rooms, x_objects,
           src_rr, dst_rr, src_ro, dst_ro, src_or, dst_or, src_oo, dst_oo,
           Wn1_rr, Wr1_rr, b1_rr, Wn1_ro, Wr1_ro, b1_ro,
           Wn1_or, Wr1_or, b1_or, Wn1_oo, Wr1_oo, b1_oo,
           Wn2_rr, Wr2_rr, b2_rr, Wn2_ro, Wr2_ro, b2_ro,
           Wn2_or, Wr2_or, b2_or, Wn2_oo, Wr2_oo, b2_oo):
    xr4 = x_rooms.reshape(NR * NCH, CW)
    xo4 = x_objects.reshape(NO * NCH, CW)

    def pad(s, d, ep, dummy):
        pe = ep - s.shape[0]
        s2 = jnp.concatenate([s, jnp.zeros((pe,), jnp.int32)])
        d2 = jnp.concatenate([d, jnp.full((pe,), dummy, jnp.int32)])
        return s2, d2

    def pad2(s_, d_, ep, dummy):
        s2, d2 = pad(s_, d_, ep, dummy)
        return s2.reshape(-1, K), d2.reshape(-1, K)

    srr, drr = pad2(src_rr, dst_rr, ERRP, NR)
    sro, dro = pad2(src_ro, dst_ro, EROP, NO)
    sor, dor = pad2(src_or, dst_or, EORP, NR)
    soo, doo = pad2(src_oo, dst_oo, EOOP, NO)

    arr, aro, aor, aoo, crr, cro, cor, coo = _sc_l1()(
        xr4, xo4, srr, drr, sro, dro, sor, dor, soo, doo)
    crr = crr[:NR].reshape(NR, 1)
    cor = cor[:NR].reshape(NR, 1)
    cro = cro[:NO].reshape(NO, 1)
    coo = coo[:NO].reshape(NO, 1)

    z2rr, root2 = _t1r()(
        arr, aor, crr, cor, x_rooms,
        Wn1_rr, Wn1_or, Wr1_rr, Wr1_or,
        b1_rr.reshape(1, -1), b1_or.reshape(1, -1),
        Wn2_rr, Wr2_rr, Wr2_or,
        b2_rr.reshape(1, -1), b2_or.reshape(1, -1))

    [z2or] = _t1o()(
        aro, aoo, cro, coo, x_objects,
        Wn1_ro, Wn1_oo, Wr1_ro, Wr1_oo,
        b1_ro.reshape(1, -1), b1_oo.reshape(1, -1),
        Wn2_or)

    a2rr, a2or = _sc_l2()(z2rr, z2or, srr, drr, sor, dor)

    [out] = _t2()(a2rr, a2or, crr, cor, root2)
    return out
